# baseline (speedup 1.0000x reference)
import functools

import jax
import jax.numpy as jnp
from jax import lax
from jax.experimental import pallas as pl
from jax.experimental.pallas import tpu as pltpu

_B, _S, _F = 8, 128, 2048
_HID = 64
_K = 8


def _fused_kernel(x_ref, w1_ref, b1_ref, w2_ref, b2_ref, out_ref,
                  h1_scr, sm_scr, rm_scr, cnt_scr):
    f = _F
    kk = min(_K, f - 1)
    i = pl.program_id(0)
    slot = lax.rem(i, 2)
    prev_slot = 1 - slot

    xb = x_ref[0]
    v = jnp.mean(xb, axis=0, keepdims=True)
    h1_scr[pl.ds(slot, 1)] = lax.dot_general(
        w1_ref[...], xb, (((0,), (0,)), ((), ())),
        preferred_element_type=jnp.float32)[None]

    norm = jnp.sqrt(jnp.sum(v * v, axis=1, keepdims=True))
    vn = v / jnp.maximum(norm, 1e-12)
    work = jnp.concatenate([vn, -vn], axis=0)
    mask = jnp.zeros((2, f), jnp.float32)
    for r in range(kk + 1):
        mx = jnp.max(work, axis=1, keepdims=True)
        sel = work == mx
        if r > 0:
            mask = jnp.where(sel, 1.0, mask)
        work = jnp.where(sel, -jnp.inf, work)
    p = mask[0:1]
    n = mask[1:2]
    a = (vn > 0).astype(jnp.float32)
    b = (vn < 0).astype(jnp.float32)
    ap, an = a * p, a * n
    bp, bn = b * p, b * n
    m = ap + bn

    def rsum(t):
        return jnp.sum(t, axis=1, keepdims=True)

    sum_rows = (p, n, a, b, ap, bp, an, bn)
    row_rows = (a, b, p, n, ap, an, bp, bn)
    cnts = [rsum(t) for t in sum_rows]
    deg = (1.0 - m
           + cnts[0] * a + cnts[1] * b + cnts[2] * p + cnts[3] * n
           - cnts[4] * ap - cnts[5] * an - cnts[6] * bp - cnts[7] * bn)
    inv_deg = 1.0 / deg
    for k in range(8):
        sm_scr[pl.ds(slot, 1), pl.ds(k, 1), :] = sum_rows[k][None]
        rm_scr[pl.ds(slot, 1), pl.ds(k, 1), :] = (row_rows[k] * inv_deg)[None]
    rm_scr[pl.ds(slot, 1), pl.ds(8, 1), :] = ((1.0 - m) * inv_deg)[None]
    cnt_scr[pl.ds(slot, 1), :] = jnp.concatenate(cnts, axis=1)

    sm = sm_scr[prev_slot]
    rm = rm_scr[prev_slot]
    w = rm[8:9]
    cnt = cnt_scr[prev_slot]
    sgn = jnp.where(lax.broadcasted_iota(jnp.int32, (1, 8), 1) < 4, 1.0, -1.0)

    def layer(h_raw, bias):
        s = lax.dot_general(h_raw, sm, (((1,), (1,)), ((), ())),
                            preferred_element_type=jnp.float32)
        coef = jnp.concatenate([(s + bias * cnt[None, :]) * sgn, bias], axis=1)
        corr = lax.dot_general(coef, rm, (((1,), (0,)), ((), ())),
                               preferred_element_type=jnp.float32)
        return h_raw * w + corr

    g1 = jnp.maximum(layer(h1_scr[prev_slot], b1_ref[...]), 0.0)
    h2 = lax.dot_general(w2_ref[...], g1, (((0,), (0,)), ((), ())),
                         preferred_element_type=jnp.float32)
    out_ref[0] = layer(h2, b2_ref[...])


@functools.partial(jax.jit, static_argnums=())
def kernel(x, W1, b1, W2, b2):
    b1c = b1.reshape(_HID, 1)
    b2c = b2.reshape(_S, 1)
    return pl.pallas_call(
        _fused_kernel,
        grid=(_B + 1,),
        in_specs=[
            pl.BlockSpec((1, _S, _F),
                         lambda i: (jnp.minimum(i, _B - 1), 0, 0)),
            pl.BlockSpec((_S, _HID), lambda i: (0, 0)),
            pl.BlockSpec((_HID, 1), lambda i: (0, 0)),
            pl.BlockSpec((_HID, _S), lambda i: (0, 0)),
            pl.BlockSpec((_S, 1), lambda i: (0, 0)),
        ],
        out_specs=pl.BlockSpec((1, _S, _F),
                               lambda i: (jnp.maximum(i - 1, 0), 0, 0)),
        out_shape=jax.ShapeDtypeStruct((_B, _S, _F), jnp.float32),
        scratch_shapes=[
            pltpu.VMEM((2, _HID, _F), jnp.float32),
            pltpu.VMEM((2, 8, _F), jnp.float32),
            pltpu.VMEM((2, 9, _F), jnp.float32),
            pltpu.VMEM((2, 8), jnp.float32),
        ],
    )(x, W1, b1c, W2, b2c)


# speedup vs baseline: 1.2464x; 1.2464x over previous
import functools

import jax
import jax.numpy as jnp
from jax import lax
from jax.experimental import pallas as pl
from jax.experimental.pallas import tpu as pltpu

_B, _S, _F = 8, 128, 2048
_HID = 64
_K = 8


def _fused_kernel(x_ref, w1_ref, b1_ref, w2_ref, b2_ref, out_ref,
                  h1_scr, v_scr, sm_scr, rm_scr, cnt_scr, obuf, sems):
    f = _F
    kk = min(_K, f - 1)
    i = pl.program_id(0)

    @pl.when(i < _B)
    def _stage_a():
        xb = x_ref[0]
        v_scr[pl.ds(i, 1), :] = jnp.mean(xb, axis=0, keepdims=True)
        h1_scr[pl.ds(i, 1)] = lax.dot_general(
            w1_ref[...], xb, (((0,), (0,)), ((), ())),
            preferred_element_type=jnp.float32)[None]

    @pl.when(i == _B)
    def _compute_masks():
        v = v_scr[...]
        norm = jnp.sqrt(jnp.sum(v * v, axis=1, keepdims=True))
        vn = v / jnp.maximum(norm, 1e-12)

        work = jnp.concatenate([vn, -vn], axis=0)
        mask = jnp.zeros((2 * _B, f), jnp.float32)
        for r in range(kk + 1):
            mx = jnp.max(work, axis=1, keepdims=True)
            sel = work == mx
            if r > 0:
                mask = jnp.where(sel, 1.0, mask)
            work = jnp.where(sel, -jnp.inf, work)
        p = mask[:_B]
        n = mask[_B:]
        a = (vn > 0).astype(jnp.float32)
        b = (vn < 0).astype(jnp.float32)
        ap, an = a * p, a * n
        bp, bn = b * p, b * n
        m = ap + bn

        def rsum(t):
            return jnp.sum(t, axis=1, keepdims=True)

        sum_rows = (p, n, a, b, ap, bp, an, bn)
        row_rows = (a, b, p, n, ap, an, bp, bn)
        cnts = [rsum(t) for t in sum_rows]
        deg = (1.0 - m
               + cnts[0] * a + cnts[1] * b + cnts[2] * p + cnts[3] * n
               - cnts[4] * ap - cnts[5] * an - cnts[6] * bp - cnts[7] * bn)
        inv_deg = 1.0 / deg
        for k in range(8):
            sm_scr[:, k, :] = sum_rows[k]
            rm_scr[:, k, :] = row_rows[k] * inv_deg
        rm_scr[:, 8, :] = (1.0 - m) * inv_deg
        cnt_scr[...] = jnp.concatenate(cnts, axis=1)

    @pl.when(i > _B)
    def _stage_c():
        pair = i - _B - 1
        par = lax.rem(pair, 2)

        @pl.when(pair >= 2)
        def _wait_prev():
            pltpu.make_async_copy(
                obuf.at[par], out_ref.at[pl.ds(2 * (pair - 2), 2)],
                sems.at[par]).wait()

        sgn = jnp.where(
            lax.broadcasted_iota(jnp.int32, (1, 8), 1) < 4, 1.0, -1.0)

        def one_batch(bidx):
            sm = sm_scr[bidx]
            rm = rm_scr[bidx]
            w = rm[8:9]
            cnt = cnt_scr[bidx]

            def layer(h_raw, bias):
                s = lax.dot_general(h_raw, sm, (((1,), (1,)), ((), ())),
                                    preferred_element_type=jnp.float32)
                coef = jnp.concatenate(
                    [(s + bias * cnt[None, :]) * sgn, bias], axis=1)
                corr = lax.dot_general(coef, rm, (((1,), (0,)), ((), ())),
                                       preferred_element_type=jnp.float32)
                return h_raw * w + corr

            g1 = jnp.maximum(layer(h1_scr[bidx], b1_ref[...]), 0.0)
            h2 = lax.dot_general(w2_ref[...], g1, (((0,), (0,)), ((), ())),
                                 preferred_element_type=jnp.float32)
            return layer(h2, b2_ref[...])

        for j in range(2):
            res = one_batch(2 * pair + j)
            obuf[pl.ds(par, 1), pl.ds(j, 1)] = res[None, None]
        pltpu.make_async_copy(obuf.at[par], out_ref.at[pl.ds(2 * pair, 2)],
                              sems.at[par]).start()

        @pl.when(pair == 3)
        def _drain_tail():
            pltpu.make_async_copy(
                obuf.at[1 - par], out_ref.at[pl.ds(2 * (pair - 1), 2)],
                sems.at[1 - par]).wait()
            pltpu.make_async_copy(
                obuf.at[par], out_ref.at[pl.ds(2 * pair, 2)],
                sems.at[par]).wait()


@functools.partial(jax.jit, static_argnums=())
def kernel(x, W1, b1, W2, b2):
    b1c = b1.reshape(_HID, 1)
    b2c = b2.reshape(_S, 1)
    out = pl.pallas_call(
        _fused_kernel,
        grid=(_B + 1 + _B // 2,),
        in_specs=[
            pl.BlockSpec((1, _S, _F),
                         lambda i: (jnp.minimum(i, _B - 1), 0, 0)),
            pl.BlockSpec((_S, _HID), lambda i: (0, 0)),
            pl.BlockSpec((_HID, 1), lambda i: (0, 0)),
            pl.BlockSpec((_HID, _S), lambda i: (0, 0)),
            pl.BlockSpec((_S, 1), lambda i: (0, 0)),
        ],
        out_specs=pl.BlockSpec(memory_space=pl.ANY),
        out_shape=jax.ShapeDtypeStruct((_B, _S, _F), jnp.float32),
        scratch_shapes=[
            pltpu.VMEM((_B, _HID, _F), jnp.float32),
            pltpu.VMEM((_B, _F), jnp.float32),
            pltpu.VMEM((_B, 8, _F), jnp.float32),
            pltpu.VMEM((_B, 9, _F), jnp.float32),
            pltpu.VMEM((_B, 8), jnp.float32),
            pltpu.VMEM((2, 2, _S, _F), jnp.float32),
            pltpu.SemaphoreType.DMA((2,)),
        ],
    )(x, W1, b1c, W2, b2c)
    return out


# speedup vs baseline: 1.3710x; 1.0999x over previous
import functools

import jax
import jax.numpy as jnp
from jax import lax
from jax.experimental import pallas as pl
from jax.experimental.pallas import tpu as pltpu

_B, _S, _F = 8, 128, 2048
_HID = 64
_K = 8


def _fused_kernel(x_ref, w1_ref, b1_ref, w2_ref, b2_ref, out_ref,
                  h1_scr, v_scr, sm_scr, rm_scr, cnt_scr, obuf, sems):
    f = _F
    kk = min(_K, f - 1)
    i = pl.program_id(0)

    @pl.when(i < _B // 2)
    def _stage_a():
        for j in range(2):
            xb = x_ref[j]
            bidx = 2 * i + j
            v_scr[pl.ds(bidx, 1), :] = jnp.mean(xb, axis=0, keepdims=True)
            h1_scr[pl.ds(bidx, 1)] = lax.dot_general(
                w1_ref[...], xb, (((0,), (0,)), ((), ())),
                preferred_element_type=jnp.float32)[None]

    @pl.when(i == _B // 2)
    def _compute_masks():
        v = v_scr[...]
        norm = jnp.sqrt(jnp.sum(v * v, axis=1, keepdims=True))
        vn = v / jnp.maximum(norm, 1e-12)

        work = jnp.concatenate([vn, -vn], axis=0)
        mask = jnp.zeros((2 * _B, f), jnp.float32)
        for r in range(kk + 1):
            mx = jnp.max(work, axis=1, keepdims=True)
            sel = work == mx
            if r > 0:
                mask = jnp.where(sel, 1.0, mask)
            work = jnp.where(sel, -jnp.inf, work)
        p = mask[:_B]
        n = mask[_B:]
        a = (vn > 0).astype(jnp.float32)
        b = (vn < 0).astype(jnp.float32)
        ap, an = a * p, a * n
        bp, bn = b * p, b * n
        m = ap + bn

        def rsum(t):
            return jnp.sum(t, axis=1, keepdims=True)

        sum_rows = (p, n, a, b, ap, bp, an, bn)
        row_rows = (a, b, p, n, ap, an, bp, bn)
        cnts = [rsum(t) for t in sum_rows]
        deg = (1.0 - m
               + cnts[0] * a + cnts[1] * b + cnts[2] * p + cnts[3] * n
               - cnts[4] * ap - cnts[5] * an - cnts[6] * bp - cnts[7] * bn)
        inv_deg = 1.0 / deg
        for k in range(8):
            sm_scr[:, k, :] = sum_rows[k]
            rm_scr[:, k, :] = row_rows[k] * inv_deg
        rm_scr[:, 8, :] = (1.0 - m) * inv_deg
        cnt_scr[...] = jnp.concatenate(cnts, axis=1)

    @pl.when(i > _B // 2)
    def _stage_c():
        pair = i - _B // 2 - 1
        par = lax.rem(pair, 2)

        @pl.when(pair >= 2)
        def _wait_prev():
            pltpu.make_async_copy(
                obuf.at[par], out_ref.at[pl.ds(2 * (pair - 2), 2)],
                sems.at[par]).wait()

        sgn = jnp.where(
            lax.broadcasted_iota(jnp.int32, (1, 8), 1) < 4, 1.0, -1.0)

        def one_batch(bidx):
            sm = sm_scr[bidx]
            rm = rm_scr[bidx]
            w = rm[8:9]
            cnt = cnt_scr[bidx]

            def layer(h_raw, bias):
                s = lax.dot_general(h_raw, sm, (((1,), (1,)), ((), ())),
                                    preferred_element_type=jnp.float32)
                coef = jnp.concatenate(
                    [(s + bias * cnt[None, :]) * sgn, bias], axis=1)
                corr = lax.dot_general(coef, rm, (((1,), (0,)), ((), ())),
                                       preferred_element_type=jnp.float32)
                return h_raw * w + corr

            g1 = jnp.maximum(layer(h1_scr[bidx], b1_ref[...]), 0.0)
            h2 = lax.dot_general(w2_ref[...], g1, (((0,), (0,)), ((), ())),
                                 preferred_element_type=jnp.float32)
            return layer(h2, b2_ref[...])

        for j in range(2):
            res = one_batch(2 * pair + j)
            obuf[pl.ds(par, 1), pl.ds(j, 1)] = res[None, None]
        pltpu.make_async_copy(obuf.at[par], out_ref.at[pl.ds(2 * pair, 2)],
                              sems.at[par]).start()

        @pl.when(pair == 3)
        def _drain_tail():
            pltpu.make_async_copy(
                obuf.at[1 - par], out_ref.at[pl.ds(2 * (pair - 1), 2)],
                sems.at[1 - par]).wait()
            pltpu.make_async_copy(
                obuf.at[par], out_ref.at[pl.ds(2 * pair, 2)],
                sems.at[par]).wait()


@functools.partial(jax.jit, static_argnums=())
def kernel(x, W1, b1, W2, b2):
    b1c = b1.reshape(_HID, 1)
    b2c = b2.reshape(_S, 1)
    out = pl.pallas_call(
        _fused_kernel,
        grid=(_B + 1,),
        in_specs=[
            pl.BlockSpec((2, _S, _F),
                         lambda i: (jnp.minimum(i, _B // 2 - 1), 0, 0)),
            pl.BlockSpec((_S, _HID), lambda i: (0, 0)),
            pl.BlockSpec((_HID, 1), lambda i: (0, 0)),
            pl.BlockSpec((_HID, _S), lambda i: (0, 0)),
            pl.BlockSpec((_S, 1), lambda i: (0, 0)),
        ],
        out_specs=pl.BlockSpec(memory_space=pl.ANY),
        out_shape=jax.ShapeDtypeStruct((_B, _S, _F), jnp.float32),
        scratch_shapes=[
            pltpu.VMEM((_B, _HID, _F), jnp.float32),
            pltpu.VMEM((_B, _F), jnp.float32),
            pltpu.VMEM((_B, 8, _F), jnp.float32),
            pltpu.VMEM((_B, 9, _F), jnp.float32),
            pltpu.VMEM((_B, 8), jnp.float32),
            pltpu.VMEM((2, 2, _S, _F), jnp.float32),
            pltpu.SemaphoreType.DMA((2,)),
        ],
    )(x, W1, b1c, W2, b2c)
    return out


# speedup vs baseline: 1.3953x; 1.0178x over previous
import functools

import jax
import jax.numpy as jnp
from jax import lax
from jax.experimental import pallas as pl
from jax.experimental.pallas import tpu as pltpu

_B, _S, _F = 8, 128, 2048
_HID = 64
_K = 8


def _fused_kernel(x_ref, w1_ref, b1_ref, w2_ref, b2_ref, out_ref,
                  h1_scr, v_scr, sm_scr, rm_scr, cnt_scr, obuf, sems):
    f = _F
    kk = min(_K, f - 1)
    i = pl.program_id(0)

    @pl.when(i < _B // 2)
    def _stage_a():
        for j in range(2):
            xb = x_ref[j]
            bidx = 2 * i + j
            v_scr[pl.ds(bidx, 1), :] = jnp.mean(xb, axis=0, keepdims=True)
            h1_scr[pl.ds(bidx, 1)] = lax.dot_general(
                w1_ref[...], xb, (((0,), (0,)), ((), ())),
                preferred_element_type=jnp.float32)[None]

    @pl.when(i == _B // 2 - 1)
    def _compute_masks():
        v = v_scr[...]
        norm = jnp.sqrt(jnp.sum(v * v, axis=1, keepdims=True))
        vn = v / jnp.maximum(norm, 1e-12)

        work = jnp.concatenate([vn, -vn], axis=0)
        mask = jnp.zeros((2 * _B, f), jnp.float32)
        for r in range(kk + 1):
            mx = jnp.max(work, axis=1, keepdims=True)
            sel = work == mx
            if r > 0:
                mask = jnp.where(sel, 1.0, mask)
            work = jnp.where(sel, -jnp.inf, work)
        p = mask[:_B]
        n = mask[_B:]
        a = (vn > 0).astype(jnp.float32)
        b = (vn < 0).astype(jnp.float32)
        ap, an = a * p, a * n
        bp, bn = b * p, b * n
        m = ap + bn

        def rsum(t):
            return jnp.sum(t, axis=1, keepdims=True)

        sum_rows = (p, n, a, b, ap, bp, an, bn)
        row_rows = (a, b, p, n, ap, an, bp, bn)
        cnts = [rsum(t) for t in sum_rows]
        deg = (1.0 - m
               + cnts[0] * a + cnts[1] * b + cnts[2] * p + cnts[3] * n
               - cnts[4] * ap - cnts[5] * an - cnts[6] * bp - cnts[7] * bn)
        inv_deg = 1.0 / deg
        for k in range(8):
            sm_scr[:, k, :] = sum_rows[k]
            rm_scr[:, k, :] = row_rows[k] * inv_deg
        rm_scr[:, 8, :] = (1.0 - m) * inv_deg
        cnt_scr[...] = jnp.concatenate(cnts, axis=1)

    @pl.when(i >= _B // 2)
    def _stage_c():
        pair = i - _B // 2
        par = lax.rem(pair, 2)

        @pl.when(pair >= 2)
        def _wait_prev():
            for jj in range(2):
                pltpu.make_async_copy(
                    obuf.at[par, jj], out_ref.at[2 * (pair - 2) + jj],
                    sems.at[par, jj]).wait()

        sgn = jnp.where(
            lax.broadcasted_iota(jnp.int32, (1, 8), 1) < 4, 1.0, -1.0)

        def one_batch(bidx):
            sm = sm_scr[bidx]
            rm = rm_scr[bidx]
            w = rm[8:9]
            cnt = cnt_scr[bidx]

            def layer(h_raw, bias):
                s = lax.dot_general(h_raw, sm, (((1,), (1,)), ((), ())),
                                    preferred_element_type=jnp.float32)
                coef = jnp.concatenate(
                    [(s + bias * cnt[None, :]) * sgn, bias], axis=1)
                corr = lax.dot_general(coef, rm, (((1,), (0,)), ((), ())),
                                       preferred_element_type=jnp.float32)
                return h_raw * w + corr

            g1 = jnp.maximum(layer(h1_scr[bidx], b1_ref[...]), 0.0)
            h2 = lax.dot_general(w2_ref[...], g1, (((0,), (0,)), ((), ())),
                                 preferred_element_type=jnp.float32)
            return layer(h2, b2_ref[...])

        for j in range(2):
            res = one_batch(2 * pair + j)
            obuf[pl.ds(par, 1), pl.ds(j, 1)] = res[None, None]
            pltpu.make_async_copy(obuf.at[par, j], out_ref.at[2 * pair + j],
                                  sems.at[par, j]).start()

        @pl.when(pair == 3)
        def _drain_tail():
            for jj in range(2):
                pltpu.make_async_copy(
                    obuf.at[1 - par, jj], out_ref.at[2 * (pair - 1) + jj],
                    sems.at[1 - par, jj]).wait()
                pltpu.make_async_copy(
                    obuf.at[par, jj], out_ref.at[2 * pair + jj],
                    sems.at[par, jj]).wait()


@functools.partial(jax.jit, static_argnums=())
def kernel(x, W1, b1, W2, b2):
    b1c = b1.reshape(_HID, 1)
    b2c = b2.reshape(_S, 1)
    out = pl.pallas_call(
        _fused_kernel,
        grid=(_B,),
        in_specs=[
            pl.BlockSpec((2, _S, _F),
                         lambda i: (jnp.minimum(i, _B // 2 - 1), 0, 0)),
            pl.BlockSpec((_S, _HID), lambda i: (0, 0)),
            pl.BlockSpec((_HID, 1), lambda i: (0, 0)),
            pl.BlockSpec((_HID, _S), lambda i: (0, 0)),
            pl.BlockSpec((_S, 1), lambda i: (0, 0)),
        ],
        out_specs=pl.BlockSpec(memory_space=pl.ANY),
        out_shape=jax.ShapeDtypeStruct((_B, _S, _F), jnp.float32),
        scratch_shapes=[
            pltpu.VMEM((_B, _HID, _F), jnp.float32),
            pltpu.VMEM((_B, _F), jnp.float32),
            pltpu.VMEM((_B, 8, _F), jnp.float32),
            pltpu.VMEM((_B, 9, _F), jnp.float32),
            pltpu.VMEM((_B, 8), jnp.float32),
            pltpu.VMEM((2, 2, _S, _F), jnp.float32),
            pltpu.SemaphoreType.DMA((2, 2)),
        ],
    )(x, W1, b1c, W2, b2c)
    return out
